# baseline (speedup 1.0000x reference)
import jax
import jax.numpy as jnp
from jax.experimental import pallas as pl

N = 10000
F = 128
C = 6
BI = 400


def _pass1_kernel(adj_ref, zeff_ref, W1_ref, b1_ref, W2_ref, m_ref):
    s = jnp.sum(adj_ref[...], axis=1, keepdims=True)
    r = jnp.dot(zeff_ref[...], W1_ref[...], preferred_element_type=jnp.float32)
    h = jax.nn.relu(s * r + b1_ref[...])
    m_ref[...] = jnp.dot(h, W2_ref[...], preferred_element_type=jnp.float32)


def _pass2_kernel(adj_ref, m_ref, b2_ref, out_ref):
    acc = jnp.dot(adj_ref[...], m_ref[...], preferred_element_type=jnp.float32)
    o = acc + b2_ref[...]
    mx = jnp.max(o, axis=1, keepdims=True)
    lse = jnp.log(jnp.sum(jnp.exp(o - mx), axis=1, keepdims=True)) + mx
    out_ref[...] = o - lse


@jax.jit
def kernel(adj, z, W1, b1, W2, b2, n_nodes):
    zero_residual = (jnp.asarray(n_nodes) - N).astype(jnp.float32)
    z_eff = z + zero_residual
    b1r = b1.reshape(1, F)
    b2r = b2.reshape(1, C)

    grid = (N // BI,)
    M = pl.pallas_call(
        _pass1_kernel,
        grid=grid,
        in_specs=[
            pl.BlockSpec((BI, N), lambda i: (i, 0)),
            pl.BlockSpec((1, F), lambda i: (0, 0)),
            pl.BlockSpec((F, F), lambda i: (0, 0)),
            pl.BlockSpec((1, F), lambda i: (0, 0)),
            pl.BlockSpec((F, C), lambda i: (0, 0)),
        ],
        out_specs=pl.BlockSpec((BI, C), lambda i: (i, 0)),
        out_shape=jax.ShapeDtypeStruct((N, C), jnp.float32),
    )(adj, z_eff, W1, b1r, W2)

    out = pl.pallas_call(
        _pass2_kernel,
        grid=grid,
        in_specs=[
            pl.BlockSpec((BI, N), lambda i: (i, 0)),
            pl.BlockSpec((N, C), lambda i: (0, 0)),
            pl.BlockSpec((1, C), lambda i: (0, 0)),
        ],
        out_specs=pl.BlockSpec((BI, C), lambda i: (i, 0)),
        out_shape=jax.ShapeDtypeStruct((N, C), jnp.float32),
    )(adj, M, b2r)
    return out
